# baseline (speedup 1.0000x reference)
import functools

import jax
import jax.numpy as jnp
from jax import lax
from jax.experimental import pallas as pl
from jax.experimental.pallas import tpu as pltpu

BN = 8192

_dot = functools.partial(lax.dot_general,
                         precision=lax.Precision.DEFAULT,
                         preferred_element_type=jnp.float32)


def _mm(a, b, ca, cb):
    return _dot(a, b, (((ca,), (cb,)), ((), ())))


def _inv3(m, safe=None):
    a, b, c, d, e, f, g, h, i = m
    det = a * e * i + b * f * g + d * h * c - g * e * c - a * h * f - d * b * i
    if safe is not None:
        det = jnp.where(safe, det, 1.0)
    r = 1.0 / det
    out = [(e * i - f * h) * r, (c * h - b * i) * r, (b * f - c * e) * r,
           (f * g - d * i) * r, (a * i - c * g) * r, (c * d - a * f) * r,
           (d * h - e * g) * r, (b * g - a * h) * r, (a * e - b * d) * r]
    if safe is not None:
        out = [jnp.where(safe, x, 0.0) for x in out]
    return out


def _mv(m, v):
    return [m[3 * i + 0] * v[0] + m[3 * i + 1] * v[1] + m[3 * i + 2] * v[2]
            for i in range(3)]


def _mtv(m, v):
    return [m[0 + i] * v[0] + m[3 + i] * v[1] + m[6 + i] * v[2]
            for i in range(3)]


def _rows(x, base, n):
    return [x[base + k:base + k + 1, :] for k in range(n)]


def _onehot(idx_ref, g):
    idxv = idx_ref[...]
    io = lax.broadcasted_iota(jnp.int32, (g, BN), 0)
    return jnp.where(io == idxv, 1.0, 0.0).astype(jnp.float32)


def _amat(s9):
    out = []
    for k in range(9):
        row = s9[k:k + 1, :]
        if k in (0, 4, 8):
            row = row + 1.0
        out.append(row)
    return out


def _pass1_body(n_total, g, t_ref, h_ref, posT_ref, epT_ref, idx_ref, wu_ref,
                t1_ref, v1_ref):
    i = pl.program_id(0)

    @pl.when(i == 0)
    def _():
        t1_ref[...] = jnp.zeros_like(t1_ref)

    o2 = _onehot(idx_ref, g)
    tn = _mm(t_ref[...], o2, 1, 0)
    e2 = jnp.exp(-2.0 * tn)
    sig = jnp.sqrt(1.0 - e2 + 1e-2)
    isig = 1.0 / sig

    h_b = h_ref[...]
    s9 = 0.05 * jnp.tanh(_mm(wu_ref[...], h_b, 0, 1))
    amat = _amat(s9)
    inv_u = [x * isig for x in _inv3(amat)]
    ep = _rows(epT_ref[...], 0, 3)
    posr = _rows(posT_ref[...], 0, 3)
    ue = [sig * (amat[3 * i0] * ep[0] + amat[3 * i0 + 1] * ep[1]
                 + amat[3 * i0 + 2] * ep[2]) for i0 in range(3)]
    y = _mtv(inv_u, ep)

    ones = jnp.ones_like(sig)
    zero = jnp.zeros_like(sig)
    vals = jnp.concatenate(
        [ones] + posr + ue + inv_u + y + [zero] * 5, axis=0)
    lane = lax.broadcasted_iota(jnp.int32, (1, BN), 1)
    mask = (lane + i * BN) < n_total
    vals = jnp.where(mask, vals, 0.0)
    v1_ref[...] = vals
    t1_ref[...] += _mm(vals, o2, 1, 1)


def _derive1_body(t_ref, tab1_ref, d1_ref):
    tr = t_ref[...]
    alpha = jnp.exp(-tr)
    e2 = jnp.exp(-2.0 * tr)
    sig = jnp.sqrt(1.0 - e2 + 1e-2)
    dsig = e2 / sig
    rr = dsig / sig
    g2 = 2.0 * tr + 0.1

    tab = tab1_ref[...]
    cnt = tab[0:1, :]
    cn = jnp.maximum(cnt, 1.0)
    ne = cnt > 0.0
    mean_pos = [tab[1 + k:2 + k, :] / cn for k in range(3)]
    mean_ue = [tab[4 + k:5 + k, :] / cn for k in range(3)]
    v = _rows(tab, 7, 9)
    ybar = _rows(tab, 16, 3)
    inv_v = _inv3(v, safe=ne)
    t1 = _mtv(inv_v, ybar)
    vt_t1 = _mtv(v, t1)
    m1 = [(vt_t1[k] - cnt * t1[k] - ybar[k]) / cn for k in range(3)]

    zero = jnp.zeros_like(tr)
    d1_ref[...] = jnp.concatenate(
        [alpha, sig, dsig, rr, g2] + mean_pos + mean_ue + t1 + m1
        + [zero] * 7, axis=0)


def _pass2_body(n_total, g, h_ref, eh_ref, v1_ref, idx_ref,
                d1_ref, wu_ref, w1_ref, w2_ref, wp_ref, t2_ref, v2_ref):
    i = pl.program_id(0)

    @pl.when(i == 0)
    def _():
        t2_ref[...] = jnp.zeros_like(t2_ref)

    o2 = _onehot(idx_ref, g)
    d1 = d1_ref[...]
    gt = _mm(d1, o2, 1, 0)
    gn = jnp.transpose(gt[0:8, :])
    al, sg = gt[0:1, :], gt[1:2, :]
    mp = _rows(gt, 5, 3)
    mu = _rows(gt, 8, 3)
    al_n = gn[:, 0:1]
    sg_n = gn[:, 1:2]

    h_b = h_ref[...]
    eh_b = eh_ref[...]
    z_h = al_n * h_b + sg_n * eh_b
    hh = _mm(jnp.tanh(_mm(z_h, w1_ref[...], 1, 0)), w2_ref[...], 1, 0)
    dsq = h_b - hh
    ssq = _mm(jnp.ones((1, dsq.shape[1]), jnp.float32), dsq * dsq, 1, 1)

    sg_safe = jnp.where(sg == 0.0, 1.0, sg)
    isig = 1.0 / sg_safe
    s2 = 0.05 * jnp.tanh(_mm(wu_ref[...], hh, 0, 1))
    a2 = _amat(s2)
    inv_u2 = [x * isig for x in _inv3(a2)]
    u2 = [x * sg for x in a2]

    v1 = v1_ref[...]
    posr = _rows(v1, 1, 3)
    ue = _rows(v1, 4, 3)
    wp = wp_ref[0]
    zc = [ue[k] - mu[k] for k in range(3)]
    pos_hat = [wp * (al * (posr[k] - mp[k]) + zc[k]) for k in range(3)]
    pos_c = [al * posr[k] + zc[k] - al * pos_hat[k] for k in range(3)]
    q = _mv(inv_u2, pos_c)
    w = _mtv(inv_u2, q)

    zero = jnp.zeros_like(sg)
    vals = jnp.concatenate(
        inv_u2 + pos_c + q + u2 + w + pos_hat + [ssq, zero], axis=0)
    lane = lax.broadcasted_iota(jnp.int32, (1, BN), 1)
    mask = (lane + i * BN) < n_total
    vals = jnp.where(mask, vals, 0.0)
    v2_ref[...] = vals
    t2_ref[...] += _mm(vals, o2, 1, 1)


def _derive2_body(n_total, d_feat, t_ref, tab1_ref, tab2_ref, d2_ref, lh_ref):
    cnt = tab1_ref[0:1, :]
    cn = jnp.maximum(cnt, 1.0)
    ne = cnt > 0.0
    tab = tab2_ref[...]
    v2 = _rows(tab, 0, 9)
    spc = _rows(tab, 9, 3)
    sq = _rows(tab, 12, 3)
    su2 = _rows(tab, 15, 9)
    sw = _rows(tab, 24, 3)
    inv_v2 = _inv3(v2, safe=ne)
    c2 = _mv(inv_v2, [sq[k] - spc[k] for k in range(3)])
    v2t_c2 = _mtv(v2, c2)
    ybar2 = [sw[k] + v2t_c2[k] for k in range(3)]
    su2_c2 = _mv(su2, c2)
    mu2e2 = [(spc[k] + su2_c2[k]) / cn for k in range(3)]
    t2 = _mtv(inv_v2, ybar2)
    v2t_t2 = _mtv(v2, t2)
    m2 = [(v2t_t2[k] - cnt * t2[k] - ybar2[k]) / cn for k in range(3)]
    zero = jnp.zeros_like(cnt)
    d2_ref[...] = jnp.concatenate(c2 + mu2e2 + t2 + m2 + [zero] * 4, axis=0)

    tr = t_ref[...]
    alpha = jnp.exp(-tr)
    e2 = jnp.exp(-2.0 * tr)
    sig = jnp.sqrt(1.0 - e2 + 1e-2)
    dsig = e2 / sig
    g2 = 2.0 * tr + 0.1
    kappa = alpha * (1.0 + (dsig + 0.5 * g2 / sig) / sig)
    ssq_g = tab[30:31, :]
    lh_ref[0] = jnp.sum(kappa * kappa / g2 * ssq_g) / (n_total * d_feat)


def _pass3_body(n_total, g, nb, v1_ref, v2_ref, idx_ref, d1_ref, d2_ref,
                lh_ref, out_ref, sm):
    i = pl.program_id(0)

    @pl.when(i == 0)
    def _():
        sm[0] = 0.0

    o2 = _onehot(idx_ref, g)
    gt = _mm(d1_ref[...], o2, 1, 0)
    g2t = _mm(d2_ref[...], o2, 1, 0)

    al = gt[0:1, :]
    rr = gt[3:4, :]
    gg = gt[4:5, :]
    mu = _rows(gt, 8, 3)
    t1v = _rows(gt, 11, 3)
    m1v = _rows(gt, 14, 3)
    c2v = _rows(g2t, 0, 3)
    mu2 = _rows(g2t, 3, 3)
    t2v = _rows(g2t, 6, 3)
    m2v = _rows(g2t, 9, 3)

    v1 = v1_ref[...]
    v2 = v2_ref[...]
    posr = _rows(v1, 1, 3)
    ue = _rows(v1, 4, 3)
    inv_u = _rows(v1, 7, 9)
    y = _rows(v1, 16, 3)
    inv_u2 = _rows(v2, 0, 9)
    pos_c = _rows(v2, 9, 3)
    w = _rows(v2, 24, 3)
    u2 = _rows(v2, 15, 9)
    pos_hat = _rows(v2, 27, 3)

    u2c2 = _mv(u2, c2v)
    iu2c2 = _mtv(inv_u2, c2v)
    iu2t2 = _mtv(inv_u2, t2v)
    iu1t1 = _mtv(inv_u, t1v)

    lane = lax.broadcasted_iota(jnp.int32, (1, BN), 1)
    mask = (lane + i * BN) < n_total
    ig = jnp.where(mask, 1.0 / jnp.where(gg == 0.0, 1.0, gg), 0.0)
    s_p = jnp.zeros((), jnp.float32)
    for k in range(3):
        u2e2 = pos_c[k] + u2c2[k]
        dz2 = -al * pos_hat[k] + rr * (u2e2 - mu2[k])
        y2 = w[k] + iu2c2[k]
        score2 = (iu2t2[k] - t2v[k]) - y2 - m2v[k]
        ap = dz2 - 0.5 * gg * score2
        score1 = (iu1t1[k] - t1v[k]) - y[k] - m1v[k]
        zc = ue[k] - mu[k]
        tgt = -al * posr[k] + rr * zc - 0.5 * gg * score1
        dd = ap - tgt
        s_p += jnp.sum(dd * dd * ig)
    sm[0] += s_p

    @pl.when(i == nb - 1)
    def _():
        out_ref[0] = lh_ref[0]
        out_ref[1] = sm[0] / (n_total * 3.0)


def kernel(t, h, pos, eps_h, eps_pos, W_u, W1, W2, w_pos, index):
    n_total, d_feat = h.shape
    g = t.shape[0]
    nb = (n_total + BN - 1) // BN
    n_pad = nb * BN

    f32 = jnp.float32
    t_row = t.reshape(1, g).astype(f32)
    idx_p = jnp.concatenate(
        [index.astype(jnp.int32),
         jnp.full((n_pad - n_total,), g, jnp.int32)]).reshape(1, n_pad)
    posT = pos.T
    epT = eps_pos.T
    wu_s = (0.02 * W_u).astype(f32)
    w1_s = (0.05 * W1).astype(f32)
    w2_s = (0.05 * W2).astype(f32)
    wp = w_pos.astype(f32)

    node_spec = pl.BlockSpec((BN, d_feat), lambda i: (i, 0))
    row3_spec = pl.BlockSpec((3, BN), lambda i: (0, i))
    idx_spec = pl.BlockSpec((1, BN), lambda i: (0, i))
    v1_spec = pl.BlockSpec((24, BN), lambda i: (0, i))
    v2_spec = pl.BlockSpec((32, BN), lambda i: (0, i))

    def full(shape):
        return pl.BlockSpec(shape, lambda *a: tuple(0 for _ in shape))

    smem_spec = pl.BlockSpec(memory_space=pltpu.MemorySpace.SMEM)

    tab1, vals1 = pl.pallas_call(
        functools.partial(_pass1_body, n_total, g),
        grid=(nb,),
        in_specs=[full((1, g)), node_spec, row3_spec, row3_spec, idx_spec,
                  full((d_feat, 9))],
        out_specs=[full((24, g)), v1_spec],
        out_shape=[jax.ShapeDtypeStruct((24, g), f32),
                   jax.ShapeDtypeStruct((24, n_pad), f32)],
    )(t_row, h, posT, epT, idx_p, wu_s)

    d1 = pl.pallas_call(
        _derive1_body,
        in_specs=[full((1, g)), full((24, g))],
        out_specs=full((24, g)),
        out_shape=jax.ShapeDtypeStruct((24, g), f32),
    )(t_row, tab1)

    tab2, vals2 = pl.pallas_call(
        functools.partial(_pass2_body, n_total, g),
        grid=(nb,),
        in_specs=[node_spec, node_spec, v1_spec, idx_spec,
                  full((24, g)), full((d_feat, 9)),
                  full((d_feat, d_feat)), full((d_feat, d_feat)), smem_spec],
        out_specs=[full((32, g)), v2_spec],
        out_shape=[jax.ShapeDtypeStruct((32, g), f32),
                   jax.ShapeDtypeStruct((32, n_pad), f32)],
    )(h, eps_h, vals1, idx_p, d1, wu_s, w1_s, w2_s, wp)

    d2, loss_h = pl.pallas_call(
        functools.partial(_derive2_body, n_total, d_feat),
        in_specs=[full((1, g)), full((24, g)), full((32, g))],
        out_specs=[full((16, g)), smem_spec],
        out_shape=[jax.ShapeDtypeStruct((16, g), f32),
                   jax.ShapeDtypeStruct((1,), f32)],
    )(t_row, tab1, tab2)

    losses = pl.pallas_call(
        functools.partial(_pass3_body, n_total, g, nb),
        grid=(nb,),
        in_specs=[v1_spec, v2_spec, idx_spec, full((24, g)), full((16, g)),
                  smem_spec],
        out_specs=smem_spec,
        out_shape=jax.ShapeDtypeStruct((2,), f32),
        scratch_shapes=[pltpu.SMEM((1,), f32)],
    )(vals1, vals2, idx_p, d1, d2, loss_h)

    return losses


# speedup vs baseline: 1.0034x; 1.0034x over previous
import functools

import jax
import jax.numpy as jnp
from jax import lax
from jax.experimental import pallas as pl
from jax.experimental.pallas import tpu as pltpu

BN = 4096

_dot = functools.partial(lax.dot_general,
                         precision=lax.Precision.DEFAULT,
                         preferred_element_type=jnp.float32)


def _mm(a, b, ca, cb):
    return _dot(a, b, (((ca,), (cb,)), ((), ())))


def _inv3(m, safe=None, extra_scale=None):
    a, b, c, d, e, f, g, h, i = m
    det = a * e * i + b * f * g + d * h * c - g * e * c - a * h * f - d * b * i
    if safe is not None:
        det = jnp.where(safe, det, 1.0)
    if extra_scale is not None:
        det = det * extra_scale
    r = 1.0 / det
    out = [(e * i - f * h) * r, (c * h - b * i) * r, (b * f - c * e) * r,
           (f * g - d * i) * r, (a * i - c * g) * r, (c * d - a * f) * r,
           (d * h - e * g) * r, (b * g - a * h) * r, (a * e - b * d) * r]
    if safe is not None:
        out = [jnp.where(safe, x, 0.0) for x in out]
    return out


def _mv(m, v):
    return [m[3 * i + 0] * v[0] + m[3 * i + 1] * v[1] + m[3 * i + 2] * v[2]
            for i in range(3)]


def _mtv(m, v):
    return [m[0 + i] * v[0] + m[3 + i] * v[1] + m[6 + i] * v[2]
            for i in range(3)]


def _rows(x, base, n):
    return [x[base + k:base + k + 1, :] for k in range(n)]


def _onehot(idx_ref, g):
    idxv = idx_ref[...]
    io = lax.broadcasted_iota(jnp.int32, (g, BN), 0)
    return jnp.where(io == idxv, 1.0, 0.0).astype(jnp.float32)


def _amat(s9):
    out = []
    for k in range(9):
        row = s9[k:k + 1, :]
        if k in (0, 4, 8):
            row = row + 1.0
        out.append(row)
    return out


def _pass1_body(n_total, g, t_ref, h_ref, posT_ref, epT_ref, idx_ref, wu_ref,
                t1_ref, v1_ref):
    i = pl.program_id(0)

    @pl.when(i == 0)
    def _():
        t1_ref[...] = jnp.zeros_like(t1_ref)

    o2 = _onehot(idx_ref, g)
    tn = _mm(t_ref[...], o2, 1, 0)
    e2 = jnp.exp(-2.0 * tn)
    sig = jnp.sqrt(1.0 - e2 + 1e-2)

    h_b = h_ref[...]
    s9 = 0.05 * jnp.tanh(_mm(wu_ref[...], h_b, 0, 1))
    amat = _amat(s9)
    inv_u = _inv3(amat, extra_scale=sig)
    ep = _rows(epT_ref[...], 0, 3)
    posr = _rows(posT_ref[...], 0, 3)
    ue = [sig * (amat[3 * i0] * ep[0] + amat[3 * i0 + 1] * ep[1]
                 + amat[3 * i0 + 2] * ep[2]) for i0 in range(3)]
    y = _mtv(inv_u, ep)

    ones = jnp.ones_like(sig)
    zero = jnp.zeros_like(sig)
    vals = jnp.concatenate(
        [ones] + posr + ue + inv_u + y + [zero] * 5, axis=0)
    lane = lax.broadcasted_iota(jnp.int32, (1, BN), 1)
    mask = (lane + i * BN) < n_total
    vals = jnp.where(mask, vals, 0.0)
    v1_ref[...] = vals
    t1_ref[...] += _mm(vals, o2, 1, 1)


def _derive1_body(t_ref, tab1_ref, d1_ref):
    tr = t_ref[...]
    alpha = jnp.exp(-tr)
    e2 = jnp.exp(-2.0 * tr)
    sig = jnp.sqrt(1.0 - e2 + 1e-2)
    dsig = e2 / sig
    rr = dsig / sig
    g2 = 2.0 * tr + 0.1

    tab = tab1_ref[...]
    cnt = tab[0:1, :]
    cn = jnp.maximum(cnt, 1.0)
    ne = cnt > 0.0
    mean_pos = [tab[1 + k:2 + k, :] / cn for k in range(3)]
    mean_ue = [tab[4 + k:5 + k, :] / cn for k in range(3)]
    v = _rows(tab, 7, 9)
    ybar = _rows(tab, 16, 3)
    inv_v = _inv3(v, safe=ne)
    t1 = _mtv(inv_v, ybar)
    vt_t1 = _mtv(v, t1)
    m1 = [(vt_t1[k] - cnt * t1[k] - ybar[k]) / cn for k in range(3)]

    zero = jnp.zeros_like(tr)
    d1_ref[...] = jnp.concatenate(
        [alpha, sig, dsig, rr, g2] + mean_pos + mean_ue + t1 + m1
        + [zero] * 7, axis=0)


def _pass2_body(n_total, g, h_ref, eh_ref, v1_ref, idx_ref,
                d1_ref, wu_ref, w1_ref, w2_ref, wp_ref, t2_ref, v2_ref):
    i = pl.program_id(0)

    @pl.when(i == 0)
    def _():
        t2_ref[...] = jnp.zeros_like(t2_ref)

    o2 = _onehot(idx_ref, g)
    d1 = d1_ref[...]
    gt = _mm(d1, o2, 1, 0)
    gn = jnp.transpose(gt[0:8, :])
    al, sg = gt[0:1, :], gt[1:2, :]
    mp = _rows(gt, 5, 3)
    mu = _rows(gt, 8, 3)
    al_n = gn[:, 0:1]
    sg_n = gn[:, 1:2]

    h_b = h_ref[...]
    eh_b = eh_ref[...]
    z_h = al_n * h_b + sg_n * eh_b
    hh = _mm(jnp.tanh(_mm(z_h, w1_ref[...], 1, 0)), w2_ref[...], 1, 0)
    dsq = h_b - hh
    ssq = _mm(jnp.ones((1, dsq.shape[1]), jnp.float32), dsq * dsq, 1, 1)

    sg_safe = jnp.where(sg == 0.0, 1.0, sg)
    s2 = 0.05 * jnp.tanh(_mm(wu_ref[...], hh, 0, 1))
    a2 = _amat(s2)
    inv_u2 = _inv3(a2, extra_scale=sg_safe)
    u2 = [x * sg for x in a2]

    v1 = v1_ref[...]
    posr = _rows(v1, 1, 3)
    ue = _rows(v1, 4, 3)
    wp = wp_ref[0]
    zc = [ue[k] - mu[k] for k in range(3)]
    pos_hat = [wp * (al * (posr[k] - mp[k]) + zc[k]) for k in range(3)]
    pos_c = [al * posr[k] + zc[k] - al * pos_hat[k] for k in range(3)]
    q = _mv(inv_u2, pos_c)
    w = _mtv(inv_u2, q)

    zero = jnp.zeros_like(sg)
    vals = jnp.concatenate(
        inv_u2 + pos_c + q + u2 + w + pos_hat + [ssq, zero], axis=0)
    lane = lax.broadcasted_iota(jnp.int32, (1, BN), 1)
    mask = (lane + i * BN) < n_total
    vals = jnp.where(mask, vals, 0.0)
    v2_ref[...] = vals
    t2_ref[...] += _mm(vals, o2, 1, 1)


def _derive2_body(n_total, d_feat, t_ref, tab1_ref, tab2_ref, d2_ref, lh_ref):
    cnt = tab1_ref[0:1, :]
    cn = jnp.maximum(cnt, 1.0)
    ne = cnt > 0.0
    tab = tab2_ref[...]
    v2 = _rows(tab, 0, 9)
    spc = _rows(tab, 9, 3)
    sq = _rows(tab, 12, 3)
    su2 = _rows(tab, 15, 9)
    sw = _rows(tab, 24, 3)
    inv_v2 = _inv3(v2, safe=ne)
    c2 = _mv(inv_v2, [sq[k] - spc[k] for k in range(3)])
    v2t_c2 = _mtv(v2, c2)
    ybar2 = [sw[k] + v2t_c2[k] for k in range(3)]
    su2_c2 = _mv(su2, c2)
    mu2e2 = [(spc[k] + su2_c2[k]) / cn for k in range(3)]
    t2 = _mtv(inv_v2, ybar2)
    v2t_t2 = _mtv(v2, t2)
    m2 = [(v2t_t2[k] - cnt * t2[k] - ybar2[k]) / cn for k in range(3)]
    zero = jnp.zeros_like(cnt)
    d2_ref[...] = jnp.concatenate(c2 + mu2e2 + t2 + m2 + [zero] * 4, axis=0)

    tr = t_ref[...]
    alpha = jnp.exp(-tr)
    e2 = jnp.exp(-2.0 * tr)
    sig = jnp.sqrt(1.0 - e2 + 1e-2)
    dsig = e2 / sig
    g2 = 2.0 * tr + 0.1
    kappa = alpha * (1.0 + (dsig + 0.5 * g2 / sig) / sig)
    ssq_g = tab[30:31, :]
    lh_ref[0] = jnp.sum(kappa * kappa / g2 * ssq_g) / (n_total * d_feat)


def _pass3_body(n_total, g, nb, v1_ref, v2_ref, idx_ref, d1_ref, d2_ref,
                lh_ref, out_ref, sm):
    i = pl.program_id(0)

    @pl.when(i == 0)
    def _():
        sm[0] = 0.0

    o2 = _onehot(idx_ref, g)
    gt = _mm(d1_ref[...], o2, 1, 0)
    g2t = _mm(d2_ref[...], o2, 1, 0)

    al = gt[0:1, :]
    rr = gt[3:4, :]
    gg = gt[4:5, :]
    mu = _rows(gt, 8, 3)
    t1v = _rows(gt, 11, 3)
    m1v = _rows(gt, 14, 3)
    c2v = _rows(g2t, 0, 3)
    mu2 = _rows(g2t, 3, 3)
    t2v = _rows(g2t, 6, 3)
    m2v = _rows(g2t, 9, 3)

    v1 = v1_ref[...]
    v2 = v2_ref[...]
    posr = _rows(v1, 1, 3)
    ue = _rows(v1, 4, 3)
    inv_u = _rows(v1, 7, 9)
    y = _rows(v1, 16, 3)
    inv_u2 = _rows(v2, 0, 9)
    pos_c = _rows(v2, 9, 3)
    w = _rows(v2, 24, 3)
    u2 = _rows(v2, 15, 9)
    pos_hat = _rows(v2, 27, 3)

    u2c2 = _mv(u2, c2v)
    iu2c2 = _mtv(inv_u2, c2v)
    iu2t2 = _mtv(inv_u2, t2v)
    iu1t1 = _mtv(inv_u, t1v)

    lane = lax.broadcasted_iota(jnp.int32, (1, BN), 1)
    mask = (lane + i * BN) < n_total
    ig = jnp.where(mask, 1.0 / jnp.where(gg == 0.0, 1.0, gg), 0.0)
    s_p = jnp.zeros((), jnp.float32)
    for k in range(3):
        u2e2 = pos_c[k] + u2c2[k]
        dz2 = -al * pos_hat[k] + rr * (u2e2 - mu2[k])
        y2 = w[k] + iu2c2[k]
        score2 = (iu2t2[k] - t2v[k]) - y2 - m2v[k]
        ap = dz2 - 0.5 * gg * score2
        score1 = (iu1t1[k] - t1v[k]) - y[k] - m1v[k]
        zc = ue[k] - mu[k]
        tgt = -al * posr[k] + rr * zc - 0.5 * gg * score1
        dd = ap - tgt
        s_p += jnp.sum(dd * dd * ig)
    sm[0] += s_p

    @pl.when(i == nb - 1)
    def _():
        out_ref[0] = lh_ref[0]
        out_ref[1] = sm[0] / (n_total * 3.0)


def kernel(t, h, pos, eps_h, eps_pos, W_u, W1, W2, w_pos, index):
    n_total, d_feat = h.shape
    g = t.shape[0]
    nb = (n_total + BN - 1) // BN
    n_pad = nb * BN

    f32 = jnp.float32
    t_row = t.reshape(1, g).astype(f32)
    idx_p = jnp.concatenate(
        [index.astype(jnp.int32),
         jnp.full((n_pad - n_total,), g, jnp.int32)]).reshape(1, n_pad)
    posT = pos.T
    epT = eps_pos.T
    wu_s = (0.02 * W_u).astype(f32)
    w1_s = (0.05 * W1).astype(f32)
    w2_s = (0.05 * W2).astype(f32)
    wp = w_pos.astype(f32)

    node_spec = pl.BlockSpec((BN, d_feat), lambda i: (i, 0))
    row3_spec = pl.BlockSpec((3, BN), lambda i: (0, i))
    idx_spec = pl.BlockSpec((1, BN), lambda i: (0, i))
    v1_spec = pl.BlockSpec((24, BN), lambda i: (0, i))
    v2_spec = pl.BlockSpec((32, BN), lambda i: (0, i))

    def full(shape):
        return pl.BlockSpec(shape, lambda *a: tuple(0 for _ in shape))

    smem_spec = pl.BlockSpec(memory_space=pltpu.MemorySpace.SMEM)

    tab1, vals1 = pl.pallas_call(
        functools.partial(_pass1_body, n_total, g),
        grid=(nb,),
        in_specs=[full((1, g)), node_spec, row3_spec, row3_spec, idx_spec,
                  full((d_feat, 9))],
        out_specs=[full((24, g)), v1_spec],
        out_shape=[jax.ShapeDtypeStruct((24, g), f32),
                   jax.ShapeDtypeStruct((24, n_pad), f32)],
    )(t_row, h, posT, epT, idx_p, wu_s)

    d1 = pl.pallas_call(
        _derive1_body,
        in_specs=[full((1, g)), full((24, g))],
        out_specs=full((24, g)),
        out_shape=jax.ShapeDtypeStruct((24, g), f32),
    )(t_row, tab1)

    tab2, vals2 = pl.pallas_call(
        functools.partial(_pass2_body, n_total, g),
        grid=(nb,),
        in_specs=[node_spec, node_spec, v1_spec, idx_spec,
                  full((24, g)), full((d_feat, 9)),
                  full((d_feat, d_feat)), full((d_feat, d_feat)), smem_spec],
        out_specs=[full((32, g)), v2_spec],
        out_shape=[jax.ShapeDtypeStruct((32, g), f32),
                   jax.ShapeDtypeStruct((32, n_pad), f32)],
    )(h, eps_h, vals1, idx_p, d1, wu_s, w1_s, w2_s, wp)

    d2, loss_h = pl.pallas_call(
        functools.partial(_derive2_body, n_total, d_feat),
        in_specs=[full((1, g)), full((24, g)), full((32, g))],
        out_specs=[full((16, g)), smem_spec],
        out_shape=[jax.ShapeDtypeStruct((16, g), f32),
                   jax.ShapeDtypeStruct((1,), f32)],
    )(t_row, tab1, tab2)

    losses = pl.pallas_call(
        functools.partial(_pass3_body, n_total, g, nb),
        grid=(nb,),
        in_specs=[v1_spec, v2_spec, idx_spec, full((24, g)), full((16, g)),
                  smem_spec],
        out_specs=smem_spec,
        out_shape=jax.ShapeDtypeStruct((2,), f32),
        scratch_shapes=[pltpu.SMEM((1,), f32)],
    )(vals1, vals2, idx_p, d1, d2, loss_h)

    return losses


# speedup vs baseline: 1.0214x; 1.0179x over previous
import functools

import jax
import jax.numpy as jnp
from jax import lax
from jax.experimental import pallas as pl
from jax.experimental.pallas import tpu as pltpu

BN = 5120

_dot = functools.partial(lax.dot_general,
                         precision=lax.Precision.DEFAULT,
                         preferred_element_type=jnp.float32)


def _mm(a, b, ca, cb):
    return _dot(a, b, (((ca,), (cb,)), ((), ())))


def _inv3(m, safe=None, extra_scale=None):
    a, b, c, d, e, f, g, h, i = m
    det = a * e * i + b * f * g + d * h * c - g * e * c - a * h * f - d * b * i
    if safe is not None:
        det = jnp.where(safe, det, 1.0)
    if extra_scale is not None:
        det = det * extra_scale
    r = 1.0 / det
    out = [(e * i - f * h) * r, (c * h - b * i) * r, (b * f - c * e) * r,
           (f * g - d * i) * r, (a * i - c * g) * r, (c * d - a * f) * r,
           (d * h - e * g) * r, (b * g - a * h) * r, (a * e - b * d) * r]
    if safe is not None:
        out = [jnp.where(safe, x, 0.0) for x in out]
    return out


def _mv(m, v):
    return [m[3 * i + 0] * v[0] + m[3 * i + 1] * v[1] + m[3 * i + 2] * v[2]
            for i in range(3)]


def _mtv(m, v):
    return [m[0 + i] * v[0] + m[3 + i] * v[1] + m[6 + i] * v[2]
            for i in range(3)]


def _rows(x, base, n):
    return [x[base + k:base + k + 1, :] for k in range(n)]


def _onehot(idx_ref, g):
    idxv = idx_ref[...]
    io = lax.broadcasted_iota(jnp.int32, (g, BN), 0)
    return (io == idxv).astype(jnp.float32)


def _amat(s9):
    out = []
    for k in range(9):
        row = s9[k:k + 1, :]
        if k in (0, 4, 8):
            row = row + 1.0
        out.append(row)
    return out


def _pass1_body(n_total, g, t_ref, h_ref, posT_ref, epT_ref, idx_ref, wu_ref,
                t1_ref, v1_ref):
    i = pl.program_id(0)

    @pl.when(i == 0)
    def _():
        t1_ref[...] = jnp.zeros_like(t1_ref)

    o2 = _onehot(idx_ref, g)
    tn = _mm(t_ref[...], o2, 1, 0)
    e2 = jnp.exp(-2.0 * tn)
    sig = jnp.sqrt(1.0 - e2 + 1e-2)

    h_b = h_ref[...]
    s9 = 0.05 * jnp.tanh(_mm(wu_ref[...], h_b, 0, 1))
    amat = _amat(s9)
    inv_u = _inv3(amat, extra_scale=sig)
    ep = _rows(epT_ref[...], 0, 3)
    posr = _rows(posT_ref[...], 0, 3)
    ue = [sig * (amat[3 * i0] * ep[0] + amat[3 * i0 + 1] * ep[1]
                 + amat[3 * i0 + 2] * ep[2]) for i0 in range(3)]
    y = _mtv(inv_u, ep)

    ones = jnp.ones_like(sig)
    zero = jnp.zeros_like(sig)
    vals = jnp.concatenate(
        [ones] + posr + ue + inv_u + y + [zero] * 5, axis=0)
    lane = lax.broadcasted_iota(jnp.int32, (1, BN), 1)
    mask = (lane + i * BN) < n_total
    vals = jnp.where(mask, vals, 0.0)
    v1_ref[...] = vals
    t1_ref[...] += _mm(vals, o2, 1, 1)


def _derive1_body(t_ref, tab1_ref, d1_ref):
    tr = t_ref[...]
    alpha = jnp.exp(-tr)
    e2 = jnp.exp(-2.0 * tr)
    sig = jnp.sqrt(1.0 - e2 + 1e-2)
    dsig = e2 / sig
    rr = dsig / sig
    g2 = 2.0 * tr + 0.1

    tab = tab1_ref[...]
    cnt = tab[0:1, :]
    cn = jnp.maximum(cnt, 1.0)
    ne = cnt > 0.0
    mean_pos = [tab[1 + k:2 + k, :] / cn for k in range(3)]
    mean_ue = [tab[4 + k:5 + k, :] / cn for k in range(3)]
    v = _rows(tab, 7, 9)
    ybar = _rows(tab, 16, 3)
    inv_v = _inv3(v, safe=ne)
    t1 = _mtv(inv_v, ybar)
    vt_t1 = _mtv(v, t1)
    m1 = [(vt_t1[k] - cnt * t1[k] - ybar[k]) / cn for k in range(3)]

    zero = jnp.zeros_like(tr)
    d1_ref[...] = jnp.concatenate(
        [alpha, sig, dsig, rr, g2] + mean_pos + mean_ue + t1 + m1
        + [zero] * 7, axis=0)


def _pass2_body(n_total, g, h_ref, eh_ref, v1_ref, idx_ref,
                d1_ref, wu_ref, w1_ref, w2_ref, wp_ref, t2_ref, v2_ref):
    i = pl.program_id(0)

    @pl.when(i == 0)
    def _():
        t2_ref[...] = jnp.zeros_like(t2_ref)

    o2 = _onehot(idx_ref, g)
    d1 = d1_ref[...]
    gt = _mm(d1, o2, 1, 0)
    gn = jnp.transpose(gt[0:8, :])
    al, sg = gt[0:1, :], gt[1:2, :]
    mp = _rows(gt, 5, 3)
    mu = _rows(gt, 8, 3)
    al_n = gn[:, 0:1]
    sg_n = gn[:, 1:2]

    h_b = h_ref[...]
    eh_b = eh_ref[...]
    z_h = al_n * h_b + sg_n * eh_b
    hh = _mm(jnp.tanh(_mm(z_h, w1_ref[...], 1, 0)), w2_ref[...], 1, 0)
    dsq = h_b - hh
    ssq = _mm(jnp.ones((1, dsq.shape[1]), jnp.float32), dsq * dsq, 1, 1)

    sg_safe = jnp.where(sg == 0.0, 1.0, sg)
    s2 = 0.05 * jnp.tanh(_mm(wu_ref[...], hh, 0, 1))
    a2 = _amat(s2)
    inv_u2 = _inv3(a2, extra_scale=sg_safe)
    u2 = [x * sg for x in a2]

    v1 = v1_ref[...]
    posr = _rows(v1, 1, 3)
    ue = _rows(v1, 4, 3)
    wp = wp_ref[0]
    zc = [ue[k] - mu[k] for k in range(3)]
    pos_hat = [wp * (al * (posr[k] - mp[k]) + zc[k]) for k in range(3)]
    pos_c = [al * posr[k] + zc[k] - al * pos_hat[k] for k in range(3)]
    q = _mv(inv_u2, pos_c)
    w = _mtv(inv_u2, q)

    zero = jnp.zeros_like(sg)
    vals = jnp.concatenate(
        inv_u2 + pos_c + q + u2 + w + pos_hat + [ssq, zero], axis=0)
    lane = lax.broadcasted_iota(jnp.int32, (1, BN), 1)
    mask = (lane + i * BN) < n_total
    vals = jnp.where(mask, vals, 0.0)
    v2_ref[...] = vals
    t2_ref[...] += _mm(vals, o2, 1, 1)


def _derive2_body(n_total, d_feat, t_ref, tab1_ref, tab2_ref, d2_ref, lh_ref):
    cnt = tab1_ref[0:1, :]
    cn = jnp.maximum(cnt, 1.0)
    ne = cnt > 0.0
    tab = tab2_ref[...]
    v2 = _rows(tab, 0, 9)
    spc = _rows(tab, 9, 3)
    sq = _rows(tab, 12, 3)
    su2 = _rows(tab, 15, 9)
    sw = _rows(tab, 24, 3)
    inv_v2 = _inv3(v2, safe=ne)
    c2 = _mv(inv_v2, [sq[k] - spc[k] for k in range(3)])
    v2t_c2 = _mtv(v2, c2)
    ybar2 = [sw[k] + v2t_c2[k] for k in range(3)]
    su2_c2 = _mv(su2, c2)
    mu2e2 = [(spc[k] + su2_c2[k]) / cn for k in range(3)]
    t2 = _mtv(inv_v2, ybar2)
    v2t_t2 = _mtv(v2, t2)
    m2 = [(v2t_t2[k] - cnt * t2[k] - ybar2[k]) / cn for k in range(3)]
    zero = jnp.zeros_like(cnt)
    d2_ref[...] = jnp.concatenate(c2 + mu2e2 + t2 + m2 + [zero] * 4, axis=0)

    tr = t_ref[...]
    alpha = jnp.exp(-tr)
    e2 = jnp.exp(-2.0 * tr)
    sig = jnp.sqrt(1.0 - e2 + 1e-2)
    dsig = e2 / sig
    g2 = 2.0 * tr + 0.1
    kappa = alpha * (1.0 + (dsig + 0.5 * g2 / sig) / sig)
    ssq_g = tab[30:31, :]
    lh_ref[0] = jnp.sum(kappa * kappa / g2 * ssq_g) / (n_total * d_feat)


def _pass3_body(n_total, g, nb, v1_ref, v2_ref, idx_ref, d1_ref, d2_ref,
                lh_ref, out_ref, sm):
    i = pl.program_id(0)

    @pl.when(i == 0)
    def _():
        sm[0] = 0.0

    o2 = _onehot(idx_ref, g)
    gt = _mm(d1_ref[...], o2, 1, 0)
    g2t = _mm(d2_ref[...], o2, 1, 0)

    al = gt[0:1, :]
    rr = gt[3:4, :]
    gg = gt[4:5, :]
    mu = _rows(gt, 8, 3)
    t1v = _rows(gt, 11, 3)
    m1v = _rows(gt, 14, 3)
    c2v = _rows(g2t, 0, 3)
    mu2 = _rows(g2t, 3, 3)
    t2v = _rows(g2t, 6, 3)
    m2v = _rows(g2t, 9, 3)

    v1 = v1_ref[...]
    v2 = v2_ref[...]
    posr = _rows(v1, 1, 3)
    ue = _rows(v1, 4, 3)
    inv_u = _rows(v1, 7, 9)
    y = _rows(v1, 16, 3)
    inv_u2 = _rows(v2, 0, 9)
    pos_c = _rows(v2, 9, 3)
    w = _rows(v2, 24, 3)
    u2 = _rows(v2, 15, 9)
    pos_hat = _rows(v2, 27, 3)

    u2c2 = _mv(u2, c2v)
    iu2c2 = _mtv(inv_u2, c2v)
    iu2t2 = _mtv(inv_u2, t2v)
    iu1t1 = _mtv(inv_u, t1v)

    lane = lax.broadcasted_iota(jnp.int32, (1, BN), 1)
    mask = (lane + i * BN) < n_total
    ig = jnp.where(mask, 1.0 / jnp.where(gg == 0.0, 1.0, gg), 0.0)
    s_p = jnp.zeros((), jnp.float32)
    for k in range(3):
        u2e2 = pos_c[k] + u2c2[k]
        dz2 = -al * pos_hat[k] + rr * (u2e2 - mu2[k])
        y2 = w[k] + iu2c2[k]
        score2 = (iu2t2[k] - t2v[k]) - y2 - m2v[k]
        ap = dz2 - 0.5 * gg * score2
        score1 = (iu1t1[k] - t1v[k]) - y[k] - m1v[k]
        zc = ue[k] - mu[k]
        tgt = -al * posr[k] + rr * zc - 0.5 * gg * score1
        dd = ap - tgt
        s_p += jnp.sum(dd * dd * ig)
    sm[0] += s_p

    @pl.when(i == nb - 1)
    def _():
        out_ref[0] = lh_ref[0]
        out_ref[1] = sm[0] / (n_total * 3.0)


def kernel(t, h, pos, eps_h, eps_pos, W_u, W1, W2, w_pos, index):
    n_total, d_feat = h.shape
    g = t.shape[0]
    nb = (n_total + BN - 1) // BN
    n_pad = nb * BN

    f32 = jnp.float32
    t_row = t.reshape(1, g).astype(f32)
    idx_p = jnp.concatenate(
        [index.astype(jnp.int32),
         jnp.full((n_pad - n_total,), g, jnp.int32)]).reshape(1, n_pad)
    posT = pos.T
    epT = eps_pos.T
    wu_s = (0.02 * W_u).astype(f32)
    w1_s = (0.05 * W1).astype(f32)
    w2_s = (0.05 * W2).astype(f32)
    wp = w_pos.astype(f32)

    node_spec = pl.BlockSpec((BN, d_feat), lambda i: (i, 0))
    row3_spec = pl.BlockSpec((3, BN), lambda i: (0, i))
    idx_spec = pl.BlockSpec((1, BN), lambda i: (0, i))
    v1_spec = pl.BlockSpec((24, BN), lambda i: (0, i))
    v2_spec = pl.BlockSpec((32, BN), lambda i: (0, i))

    def full(shape):
        return pl.BlockSpec(shape, lambda *a: tuple(0 for _ in shape))

    smem_spec = pl.BlockSpec(memory_space=pltpu.MemorySpace.SMEM)

    tab1, vals1 = pl.pallas_call(
        functools.partial(_pass1_body, n_total, g),
        grid=(nb,),
        in_specs=[full((1, g)), node_spec, row3_spec, row3_spec, idx_spec,
                  full((d_feat, 9))],
        out_specs=[full((24, g)), v1_spec],
        out_shape=[jax.ShapeDtypeStruct((24, g), f32),
                   jax.ShapeDtypeStruct((24, n_pad), f32)],
    )(t_row, h, posT, epT, idx_p, wu_s)

    d1 = pl.pallas_call(
        _derive1_body,
        in_specs=[full((1, g)), full((24, g))],
        out_specs=full((24, g)),
        out_shape=jax.ShapeDtypeStruct((24, g), f32),
    )(t_row, tab1)

    tab2, vals2 = pl.pallas_call(
        functools.partial(_pass2_body, n_total, g),
        grid=(nb,),
        in_specs=[node_spec, node_spec, v1_spec, idx_spec,
                  full((24, g)), full((d_feat, 9)),
                  full((d_feat, d_feat)), full((d_feat, d_feat)), smem_spec],
        out_specs=[full((32, g)), v2_spec],
        out_shape=[jax.ShapeDtypeStruct((32, g), f32),
                   jax.ShapeDtypeStruct((32, n_pad), f32)],
    )(h, eps_h, vals1, idx_p, d1, wu_s, w1_s, w2_s, wp)

    d2, loss_h = pl.pallas_call(
        functools.partial(_derive2_body, n_total, d_feat),
        in_specs=[full((1, g)), full((24, g)), full((32, g))],
        out_specs=[full((16, g)), smem_spec],
        out_shape=[jax.ShapeDtypeStruct((16, g), f32),
                   jax.ShapeDtypeStruct((1,), f32)],
    )(t_row, tab1, tab2)

    losses = pl.pallas_call(
        functools.partial(_pass3_body, n_total, g, nb),
        grid=(nb,),
        in_specs=[v1_spec, v2_spec, idx_spec, full((24, g)), full((16, g)),
                  smem_spec],
        out_specs=smem_spec,
        out_shape=jax.ShapeDtypeStruct((2,), f32),
        scratch_shapes=[pltpu.SMEM((1,), f32)],
    )(vals1, vals2, idx_p, d1, d2, loss_h)

    return losses
